# Initial kernel scaffold; baseline (speedup 1.0000x reference)
import jax
import jax.numpy as jnp
from jax.experimental import pallas as pl


def kernel(x, edge_index, batch, W1, b1, W2, b2, W3, b3, W4, b4, W_ih, W_hh, b_ih, b_hh, W_fc, b_fc):
    raise NotImplementedError("write your pallas kernel here")


# speedup vs baseline: 21.4513x; 21.4513x over previous
import jax
import jax.numpy as jnp
from jax import lax
from jax.experimental import pallas as pl
from jax.experimental.pallas import tpu as pltpu
from jax.experimental.pallas import tpu_sc as plsc

N = 10000
E = 320000
NUM_GRAPHS = 16
LSTM_H = 128

NC = 2
NS = 16
NW = NC * NS

CHUNK = 128
C_E = 80
C_F = 160
EP = NW * C_E * CHUNK
NR = 10240
RPT = NR // NS
TRASH = N

_SC_PARAMS = pltpu.CompilerParams(use_tc_tiling_on_sc=False)


def _sc_mesh():
    return plsc.VectorSubcoreMesh(core_axis_name="c", subcore_axis_name="s")


def _make_propagate(F, feature_split):
    C = C_F if feature_split else C_E

    def body(xs_hbm, src_hbm, dst_hbm, out_hbm, src_v, dst_v, rows0, rows1,
             acc, sg0, sg1):
        cid = lax.axis_index("c")
        sid = lax.axis_index("s")

        def zfill(i, _):
            for jj in range(F // 16):
                rows0[i, pl.ds(jj * 16, 16)] = jnp.zeros((16,), jnp.float32)
            return 0

        lax.fori_loop(0, CHUNK, zfill, 0, unroll=False)
        r0 = sid * RPT
        for z in range(RPT // CHUNK):
            pltpu.sync_copy(rows0, acc.at[pl.ds(r0 + z * CHUNK, CHUNK)])

        if feature_split:
            pltpu.sync_copy(src_hbm.at[cid, sid], src_v)
            pltpu.sync_copy(dst_hbm.at[sid], dst_v)
        else:
            w = cid * NS + sid
            pltpu.sync_copy(src_hbm.at[w], src_v)
            pltpu.sync_copy(dst_hbm.at[w], dst_v)

        rows = (rows0, rows1)
        sems = (sg0, sg1)
        pltpu.async_copy(xs_hbm.at[src_v.at[0]], rows0, sg0)
        plsc.subcore_barrier()

        def step(jj, _):
            for b in range(2):
                j = jj * 2 + b
                pltpu.make_async_copy(
                    xs_hbm.at[src_v.at[j]], rows[b], sems[b]).wait()
                pltpu.async_copy(
                    xs_hbm.at[src_v.at[j + 1]], rows[1 - b], sems[1 - b])
                pltpu.sync_copy(rows[b], acc.at[dst_v.at[j]], add=True)
            return 0

        lax.fori_loop(0, C // 2, step, 0, unroll=False)
        pltpu.make_async_copy(xs_hbm.at[src_v.at[C]], rows[0], sems[0]).wait()

        plsc.subcore_barrier()
        off = cid * NR + sid * RPT
        pltpu.sync_copy(acc.at[pl.ds(sid * RPT, RPT)],
                        out_hbm.at[pl.ds(off, RPT)])

    return pl.kernel(
        body,
        out_type=jax.ShapeDtypeStruct((NC * NR, F), jnp.float32),
        mesh=_sc_mesh(),
        compiler_params=_SC_PARAMS,
        scratch_types=[
            pltpu.VMEM((C + 1, CHUNK), jnp.int32),
            pltpu.VMEM((C, CHUNK), jnp.int32),
            pltpu.VMEM((CHUNK, F), jnp.float32),
            pltpu.VMEM((CHUNK, F), jnp.float32),
            pltpu.VMEM_SHARED((NR, F), jnp.float32),
            pltpu.SemaphoreType.DMA,
            pltpu.SemaphoreType.DMA,
        ],
    )


def _make_degree():
    def body(dst_hbm, out_hbm, dst_v, ones_v, zeros_v, acc):
        cid = lax.axis_index("c")
        sid = lax.axis_index("s")
        w = cid * NS + sid

        def fill(i, _):
            ones_v[pl.ds(i * 16, 16)] = jnp.full((16,), 1.0, jnp.float32)
            zeros_v[pl.ds(i * 16, 16)] = jnp.zeros((16,), jnp.float32)
            return 0

        lax.fori_loop(0, CHUNK // 16, fill, 0, unroll=False)
        r0 = sid * RPT
        for z in range(RPT // CHUNK):
            pltpu.sync_copy(zeros_v, acc.at[pl.ds(r0 + z * CHUNK, CHUNK)])
        pltpu.sync_copy(dst_hbm.at[w], dst_v)
        plsc.subcore_barrier()

        def step(j, _):
            pltpu.sync_copy(ones_v, acc.at[dst_v.at[j]], add=True)
            return 0

        lax.fori_loop(0, C_E, step, 0, unroll=False)
        plsc.subcore_barrier()
        off = cid * NR + sid * RPT
        pltpu.sync_copy(acc.at[pl.ds(sid * RPT, RPT)],
                        out_hbm.at[pl.ds(off, RPT)])

    return pl.kernel(
        body,
        out_type=jax.ShapeDtypeStruct((NC * NR,), jnp.float32),
        mesh=_sc_mesh(),
        compiler_params=_SC_PARAMS,
        scratch_types=[
            pltpu.VMEM((C_E, CHUNK), jnp.int32),
            pltpu.VMEM((CHUNK,), jnp.float32),
            pltpu.VMEM((CHUNK,), jnp.float32),
            pltpu.VMEM_SHARED((NR,), jnp.float32),
        ],
    )


_BLK = 2000
_GRID = N // _BLK


def _tc1_body(d0_ref, d1_ref, x_ref, dinv_ref, xs1p_ref):
    deg = d0_ref[...] + d1_ref[...] + 1.0
    dinv = lax.rsqrt(deg)
    dinv_ref[...] = dinv
    xs1 = dinv * x_ref[...]
    xs1p_ref[0] = xs1[:, :64]
    xs1p_ref[1] = xs1[:, 64:]


def _tc1(d0, d1, x):
    return pl.pallas_call(
        _tc1_body,
        grid=(_GRID,),
        in_specs=[
            pl.BlockSpec((_BLK, 1), lambda i: (i, 0)),
            pl.BlockSpec((_BLK, 1), lambda i: (i, 0)),
            pl.BlockSpec((_BLK, 128), lambda i: (i, 0)),
        ],
        out_specs=[
            pl.BlockSpec((_BLK, 1), lambda i: (i, 0)),
            pl.BlockSpec((2, _BLK, 64), lambda i: (0, i, 0)),
        ],
        out_shape=[
            jax.ShapeDtypeStruct((N, 1), jnp.float32),
            jax.ShapeDtypeStruct((2, N, 64), jnp.float32),
        ],
    )(d0, d1, x)


def _tc2_body(p0_ref, p1_ref, x0_ref, x1_ref, dinv_ref, W1_ref, b1_ref,
              W2_ref, xs2p_ref):
    dinv = dinv_ref[...]
    y1 = dinv * jnp.concatenate(
        [p0_ref[...] + x0_ref[...], p1_ref[...] + x1_ref[...]], axis=1)
    h1 = jnp.maximum(
        jnp.dot(y1, W1_ref[...], preferred_element_type=jnp.float32)
        + b1_ref[...], 0.0)
    xs2 = dinv * jnp.dot(h1, W2_ref[...], preferred_element_type=jnp.float32)
    xs2p_ref[0] = xs2[:, :64]
    xs2p_ref[1] = xs2[:, 64:]


def _tc2(p0, p1, x0, x1, dinv, W1, b1, W2):
    return pl.pallas_call(
        _tc2_body,
        grid=(_GRID,),
        in_specs=[
            pl.BlockSpec((_BLK, 64), lambda i: (i, 0)),
            pl.BlockSpec((_BLK, 64), lambda i: (i, 0)),
            pl.BlockSpec((_BLK, 64), lambda i: (i, 0)),
            pl.BlockSpec((_BLK, 64), lambda i: (i, 0)),
            pl.BlockSpec((_BLK, 1), lambda i: (i, 0)),
            pl.BlockSpec((128, 256), lambda i: (0, 0)),
            pl.BlockSpec((1, 256), lambda i: (0, 0)),
            pl.BlockSpec((256, 128), lambda i: (0, 0)),
        ],
        out_specs=pl.BlockSpec((2, _BLK, 64), lambda i: (0, i, 0)),
        out_shape=jax.ShapeDtypeStruct((2, N, 64), jnp.float32),
    )(p0, p1, x0, x1, dinv, W1, b1, W2)


def _tc3_body(p0_ref, p1_ref, x0_ref, x1_ref, dinv_ref, b2_ref, W3_ref,
              xs3_ref):
    dinv = dinv_ref[...]
    h2 = jnp.maximum(dinv * jnp.concatenate(
        [p0_ref[...] + x0_ref[...], p1_ref[...] + x1_ref[...]], axis=1)
        + b2_ref[...], 0.0)
    xs3_ref[...] = dinv * jnp.dot(h2, W3_ref[...],
                                  preferred_element_type=jnp.float32)


def _tc3(p0, p1, x0, x1, dinv, b2, W3):
    return pl.pallas_call(
        _tc3_body,
        grid=(_GRID,),
        in_specs=[
            pl.BlockSpec((_BLK, 64), lambda i: (i, 0)),
            pl.BlockSpec((_BLK, 64), lambda i: (i, 0)),
            pl.BlockSpec((_BLK, 64), lambda i: (i, 0)),
            pl.BlockSpec((_BLK, 64), lambda i: (i, 0)),
            pl.BlockSpec((_BLK, 1), lambda i: (i, 0)),
            pl.BlockSpec((1, 128), lambda i: (0, 0)),
            pl.BlockSpec((128, 64), lambda i: (0, 0)),
        ],
        out_specs=pl.BlockSpec((_BLK, 64), lambda i: (i, 0)),
        out_shape=jax.ShapeDtypeStruct((N, 64), jnp.float32),
    )(p0, p1, x0, x1, dinv, b2, W3)


def _tc4_body(p0_ref, p1_ref, xs3_ref, dinv_ref, b3_ref, W4_ref, xs4_ref):
    dinv = dinv_ref[...]
    h3 = jnp.maximum(dinv * (p0_ref[...] + p1_ref[...] + xs3_ref[...])
                     + b3_ref[...], 0.0)
    xs4_ref[...] = dinv * jnp.dot(h3, W4_ref[...],
                                  preferred_element_type=jnp.float32)


def _tc4(p0, p1, xs3, dinv, b3, W4):
    return pl.pallas_call(
        _tc4_body,
        grid=(_GRID,),
        in_specs=[
            pl.BlockSpec((_BLK, 64), lambda i: (i, 0)),
            pl.BlockSpec((_BLK, 64), lambda i: (i, 0)),
            pl.BlockSpec((_BLK, 64), lambda i: (i, 0)),
            pl.BlockSpec((_BLK, 1), lambda i: (i, 0)),
            pl.BlockSpec((1, 64), lambda i: (0, 0)),
            pl.BlockSpec((64, 32), lambda i: (0, 0)),
        ],
        out_specs=pl.BlockSpec((_BLK, 32), lambda i: (i, 0)),
        out_shape=jax.ShapeDtypeStruct((N, 32), jnp.float32),
    )(p0, p1, xs3, dinv, b3, W4)


def _tc5_body(p0_ref, p1_ref, xs4_ref, dinv_ref, b4_ref, batch_ref,
              W_ihT_ref, bg_ref, W_fc_ref, b_fc_ref, out_ref, ssum, cnt):
    i = pl.program_id(0)

    @pl.when(i == 0)
    def _():
        ssum[...] = jnp.zeros_like(ssum)
        cnt[...] = jnp.zeros_like(cnt)

    h4 = jnp.maximum(
        dinv_ref[...] * (p0_ref[...] + p1_ref[...] + xs4_ref[...])
        + b4_ref[...], 0.0)
    gids = batch_ref[...][:, 0]
    onehot = (gids[None, :] ==
              lax.broadcasted_iota(jnp.int32, (NUM_GRAPHS, _BLK), 0)
              ).astype(jnp.float32)
    ssum[...] += jnp.dot(onehot, h4, preferred_element_type=jnp.float32)
    cnt[...] += jnp.sum(onehot, axis=1, keepdims=True)

    @pl.when(i == _GRID - 1)
    def _():
        emb = ssum[...] / jnp.maximum(cnt[...], 1.0)
        gates = jnp.dot(emb, W_ihT_ref[...],
                        preferred_element_type=jnp.float32) + bg_ref[...]
        i_g = gates[:, 0 * LSTM_H:1 * LSTM_H]
        g_g = gates[:, 2 * LSTM_H:3 * LSTM_H]
        o_g = gates[:, 3 * LSTM_H:4 * LSTM_H]
        c1 = jax.nn.sigmoid(i_g) * jnp.tanh(g_g)
        h1 = jax.nn.sigmoid(o_g) * jnp.tanh(c1)
        out_ref[...] = jnp.dot(h1, W_fc_ref[...],
                               preferred_element_type=jnp.float32) + b_fc_ref[...]


def _tc5(p0, p1, xs4, dinv, b4, batch2d, W_ihT, bg, W_fc, b_fc):
    return pl.pallas_call(
        _tc5_body,
        grid=(_GRID,),
        in_specs=[
            pl.BlockSpec((_BLK, 32), lambda i: (i, 0)),
            pl.BlockSpec((_BLK, 32), lambda i: (i, 0)),
            pl.BlockSpec((_BLK, 32), lambda i: (i, 0)),
            pl.BlockSpec((_BLK, 1), lambda i: (i, 0)),
            pl.BlockSpec((1, 32), lambda i: (0, 0)),
            pl.BlockSpec((_BLK, 1), lambda i: (i, 0)),
            pl.BlockSpec((32, 4 * LSTM_H), lambda i: (0, 0)),
            pl.BlockSpec((1, 4 * LSTM_H), lambda i: (0, 0)),
            pl.BlockSpec((LSTM_H, 8), lambda i: (0, 0)),
            pl.BlockSpec((1, 8), lambda i: (0, 0)),
        ],
        out_specs=pl.BlockSpec((NUM_GRAPHS, 8), lambda i: (0, 0)),
        out_shape=jax.ShapeDtypeStruct((NUM_GRAPHS, 8), jnp.float32),
        scratch_shapes=[
            pltpu.VMEM((NUM_GRAPHS, 32), jnp.float32),
            pltpu.VMEM((NUM_GRAPHS, 1), jnp.float32),
        ],
    )(p0, p1, xs4, dinv, b4, batch2d, W_ihT, bg, W_fc, b_fc)


def kernel(x, edge_index, batch, W1, b1, W2, b2, W3, b3, W4, b4,
           W_ih, W_hh, b_ih, b_hh, W_fc, b_fc):
    src = edge_index[0].astype(jnp.int32)
    dst = edge_index[1].astype(jnp.int32)

    npad = EP - E
    fill = jnp.arange(npad, dtype=jnp.int32)
    src_p = jnp.concatenate([src, fill % 128])
    dst_p = jnp.concatenate([dst, TRASH + (fill % 16)])
    dummy = jnp.arange(CHUNK, dtype=jnp.int32)

    esrc = jnp.concatenate(
        [src_p.reshape(NW, C_E, CHUNK),
         jnp.broadcast_to(dummy, (NW, 1, CHUNK))], axis=1)
    edst = dst_p.reshape(NW, C_E, CHUNK)
    fsrc = jnp.concatenate(
        [src_p.reshape(NS, C_F, CHUNK),
         jnp.broadcast_to(dummy, (NS, 1, CHUNK))], axis=1)
    fsrc2 = jnp.stack([fsrc, fsrc + N])
    fdst = dst_p.reshape(NS, C_F, CHUNK)

    degs = _make_degree()(edst)
    d0 = degs[:N, None]
    d1 = degs[NR:NR + N, None]

    dinv, xs1p = _tc1(d0, d1, x)

    propf = _make_propagate(64, True)
    prop3 = _make_propagate(64, False)
    prop4 = _make_propagate(32, False)

    p = propf(xs1p.reshape(2 * N, 64), fsrc2, fdst)
    xs2p = _tc2(p[:N], p[NR:NR + N], xs1p[0], xs1p[1], dinv, W1, b1[None, :], W2)

    p = propf(xs2p.reshape(2 * N, 64), fsrc2, fdst)
    xs3 = _tc3(p[:N], p[NR:NR + N], xs2p[0], xs2p[1], dinv, b2[None, :], W3)

    p = prop3(xs3, esrc, edst)
    xs4 = _tc4(p[:N], p[NR:NR + N], xs3, dinv, b3[None, :], W4)

    p = prop4(xs4, esrc, edst)

    bg = (b_ih + b_hh)[None, :]
    out = _tc5(p[:N], p[NR:NR + N], xs4, dinv, b4[None, :],
               batch.astype(jnp.int32)[:, None], W_ih.T, bg, W_fc, b_fc[None, :])
    return out


# speedup vs baseline: 28.6225x; 1.3343x over previous
import jax
import jax.numpy as jnp
from jax import lax
from jax.experimental import pallas as pl
from jax.experimental.pallas import tpu as pltpu
from jax.experimental.pallas import tpu_sc as plsc

N = 10000
E = 320000
NUM_GRAPHS = 16
LSTM_H = 128

NC = 2
NS = 16
NW = NC * NS

CHUNK = 128
C_E = 80
C_F = 160
EP = NW * C_E * CHUNK
NR = 10240
RPT = NR // NS
TRASH = N

_SC_PARAMS = pltpu.CompilerParams(use_tc_tiling_on_sc=False)


def _sc_mesh():
    return plsc.VectorSubcoreMesh(core_axis_name="c", subcore_axis_name="s")


def _make_propagate(F, feature_split):
    C = C_F if feature_split else C_E

    def body(xs_hbm, src_hbm, dst_hbm, out_hbm, src_v, dst_v,
             rows0, rows1, rows2, rows3, acc,
             g0, g1, g2, g3, s0, s1, s2, s3):
        cid = lax.axis_index("c")
        sid = lax.axis_index("s")
        rows = (rows0, rows1, rows2, rows3)
        gsem = (g0, g1, g2, g3)
        ssem = (s0, s1, s2, s3)

        def zfill(i, _):
            for jj in range(F // 16):
                rows0[i, pl.ds(jj * 16, 16)] = jnp.zeros((16,), jnp.float32)
            return 0

        lax.fori_loop(0, CHUNK, zfill, 0, unroll=False)
        r0 = sid * RPT
        for z in range(RPT // CHUNK):
            pltpu.sync_copy(rows0, acc.at[pl.ds(r0 + z * CHUNK, CHUNK)])

        if feature_split:
            pltpu.sync_copy(src_hbm.at[cid, sid], src_v)
            pltpu.sync_copy(dst_hbm.at[sid], dst_v)
        else:
            w = cid * NS + sid
            pltpu.sync_copy(src_hbm.at[w], src_v)
            pltpu.sync_copy(dst_hbm.at[w], dst_v)

        pltpu.async_copy(xs_hbm.at[src_v.at[0]], rows0, g0)
        pltpu.async_copy(xs_hbm.at[src_v.at[1]], rows1, g1)
        plsc.subcore_barrier()

        def step(jj, _):
            for b in range(4):
                j = jj * 4 + b
                b2 = (b + 2) % 4
                pltpu.make_async_copy(
                    xs_hbm.at[src_v.at[j]], rows[b], gsem[b]).wait()
                pltpu.async_copy(rows[b], acc.at[dst_v.at[j]], ssem[b],
                                 add=True)

                @pl.when(j >= 2)
                def _():
                    pltpu.make_async_copy(
                        rows[b2], acc.at[dst_v.at[0]], ssem[b2]).wait()

                pltpu.async_copy(xs_hbm.at[src_v.at[j + 2]], rows[b2],
                                 gsem[b2])
            return 0

        lax.fori_loop(0, C // 4, step, 0, unroll=False)
        pltpu.make_async_copy(rows[2], acc.at[dst_v.at[0]], ssem[2]).wait()
        pltpu.make_async_copy(rows[3], acc.at[dst_v.at[0]], ssem[3]).wait()
        pltpu.make_async_copy(xs_hbm.at[src_v.at[0]], rows[0], gsem[0]).wait()
        pltpu.make_async_copy(xs_hbm.at[src_v.at[1]], rows[1], gsem[1]).wait()

        plsc.subcore_barrier()
        off = cid * NR + sid * RPT
        pltpu.sync_copy(acc.at[pl.ds(sid * RPT, RPT)],
                        out_hbm.at[pl.ds(off, RPT)])

    return pl.kernel(
        body,
        out_type=jax.ShapeDtypeStruct((NC * NR, F), jnp.float32),
        mesh=_sc_mesh(),
        compiler_params=_SC_PARAMS,
        scratch_types=[
            pltpu.VMEM((C + 2, CHUNK), jnp.int32),
            pltpu.VMEM((C, CHUNK), jnp.int32),
            pltpu.VMEM((CHUNK, F), jnp.float32),
            pltpu.VMEM((CHUNK, F), jnp.float32),
            pltpu.VMEM((CHUNK, F), jnp.float32),
            pltpu.VMEM((CHUNK, F), jnp.float32),
            pltpu.VMEM_SHARED((NR, F), jnp.float32),
            pltpu.SemaphoreType.DMA,
            pltpu.SemaphoreType.DMA,
            pltpu.SemaphoreType.DMA,
            pltpu.SemaphoreType.DMA,
            pltpu.SemaphoreType.DMA,
            pltpu.SemaphoreType.DMA,
            pltpu.SemaphoreType.DMA,
            pltpu.SemaphoreType.DMA,
        ],
    )


def _make_degree():
    def body(dst_hbm, out_hbm, dst_v, ones_v, zeros_v, acc):
        cid = lax.axis_index("c")
        sid = lax.axis_index("s")
        w = cid * NS + sid

        def fill(i, _):
            ones_v[pl.ds(i * 16, 16)] = jnp.full((16,), 1.0, jnp.float32)
            zeros_v[pl.ds(i * 16, 16)] = jnp.zeros((16,), jnp.float32)
            return 0

        lax.fori_loop(0, CHUNK // 16, fill, 0, unroll=False)
        r0 = sid * RPT
        for z in range(RPT // CHUNK):
            pltpu.sync_copy(zeros_v, acc.at[pl.ds(r0 + z * CHUNK, CHUNK)])
        pltpu.sync_copy(dst_hbm.at[w], dst_v)
        plsc.subcore_barrier()

        def step(j, _):
            pltpu.sync_copy(ones_v, acc.at[dst_v.at[j]], add=True)
            return 0

        lax.fori_loop(0, C_E, step, 0, unroll=False)
        plsc.subcore_barrier()
        off = cid * NR + sid * RPT
        pltpu.sync_copy(acc.at[pl.ds(sid * RPT, RPT)],
                        out_hbm.at[pl.ds(off, RPT)])

    return pl.kernel(
        body,
        out_type=jax.ShapeDtypeStruct((NC * NR,), jnp.float32),
        mesh=_sc_mesh(),
        compiler_params=_SC_PARAMS,
        scratch_types=[
            pltpu.VMEM((C_E, CHUNK), jnp.int32),
            pltpu.VMEM((CHUNK,), jnp.float32),
            pltpu.VMEM((CHUNK,), jnp.float32),
            pltpu.VMEM_SHARED((NR,), jnp.float32),
        ],
    )


_BLK = 2000
_GRID = N // _BLK


def _tc1_body(d0_ref, d1_ref, x_ref, dinv_ref, xs1p_ref):
    deg = d0_ref[...] + d1_ref[...] + 1.0
    dinv = lax.rsqrt(deg)
    dinv_ref[...] = dinv
    xs1 = dinv * x_ref[...]
    xs1p_ref[0] = xs1[:, :64]
    xs1p_ref[1] = xs1[:, 64:]


def _tc1(d0, d1, x):
    return pl.pallas_call(
        _tc1_body,
        grid=(_GRID,),
        in_specs=[
            pl.BlockSpec((_BLK, 1), lambda i: (i, 0)),
            pl.BlockSpec((_BLK, 1), lambda i: (i, 0)),
            pl.BlockSpec((_BLK, 128), lambda i: (i, 0)),
        ],
        out_specs=[
            pl.BlockSpec((_BLK, 1), lambda i: (i, 0)),
            pl.BlockSpec((2, _BLK, 64), lambda i: (0, i, 0)),
        ],
        out_shape=[
            jax.ShapeDtypeStruct((N, 1), jnp.float32),
            jax.ShapeDtypeStruct((2, N, 64), jnp.float32),
        ],
    )(d0, d1, x)


def _tc2_body(p0_ref, p1_ref, x0_ref, x1_ref, dinv_ref, W1_ref, b1_ref,
              W2_ref, xs2p_ref):
    dinv = dinv_ref[...]
    y1 = dinv * jnp.concatenate(
        [p0_ref[0] + x0_ref[...], p1_ref[0] + x1_ref[...]], axis=1)
    h1 = jnp.maximum(
        jnp.dot(y1, W1_ref[...], preferred_element_type=jnp.float32)
        + b1_ref[...], 0.0)
    xs2 = dinv * jnp.dot(h1, W2_ref[...], preferred_element_type=jnp.float32)
    xs2p_ref[0] = xs2[:, :64]
    xs2p_ref[1] = xs2[:, 64:]


def _tc2(p0, p1, x0, x1, dinv, W1, b1, W2):
    return pl.pallas_call(
        _tc2_body,
        grid=(_GRID,),
        in_specs=[
            pl.BlockSpec((1, _BLK, 64), lambda i: (0, i, 0)),
            pl.BlockSpec((1, _BLK, 64), lambda i: (1, i, 0)),
            pl.BlockSpec((_BLK, 64), lambda i: (i, 0)),
            pl.BlockSpec((_BLK, 64), lambda i: (i, 0)),
            pl.BlockSpec((_BLK, 1), lambda i: (i, 0)),
            pl.BlockSpec((128, 256), lambda i: (0, 0)),
            pl.BlockSpec((1, 256), lambda i: (0, 0)),
            pl.BlockSpec((256, 128), lambda i: (0, 0)),
        ],
        out_specs=pl.BlockSpec((2, _BLK, 64), lambda i: (0, i, 0)),
        out_shape=jax.ShapeDtypeStruct((2, N, 64), jnp.float32),
    )(p0, p1, x0, x1, dinv, W1, b1, W2)


def _tc3_body(p0_ref, p1_ref, x0_ref, x1_ref, dinv_ref, b2_ref, W3_ref,
              xs3_ref):
    dinv = dinv_ref[...]
    h2 = jnp.maximum(dinv * jnp.concatenate(
        [p0_ref[0] + x0_ref[...], p1_ref[0] + x1_ref[...]], axis=1)
        + b2_ref[...], 0.0)
    xs3_ref[...] = dinv * jnp.dot(h2, W3_ref[...],
                                  preferred_element_type=jnp.float32)


def _tc3(p0, p1, x0, x1, dinv, b2, W3):
    return pl.pallas_call(
        _tc3_body,
        grid=(_GRID,),
        in_specs=[
            pl.BlockSpec((1, _BLK, 64), lambda i: (0, i, 0)),
            pl.BlockSpec((1, _BLK, 64), lambda i: (1, i, 0)),
            pl.BlockSpec((_BLK, 64), lambda i: (i, 0)),
            pl.BlockSpec((_BLK, 64), lambda i: (i, 0)),
            pl.BlockSpec((_BLK, 1), lambda i: (i, 0)),
            pl.BlockSpec((1, 128), lambda i: (0, 0)),
            pl.BlockSpec((128, 64), lambda i: (0, 0)),
        ],
        out_specs=pl.BlockSpec((_BLK, 64), lambda i: (i, 0)),
        out_shape=jax.ShapeDtypeStruct((N, 64), jnp.float32),
    )(p0, p1, x0, x1, dinv, b2, W3)


def _tc4_body(p0_ref, p1_ref, xs3_ref, dinv_ref, b3_ref, W4_ref, xs4_ref):
    dinv = dinv_ref[...]
    h3 = jnp.maximum(dinv * (p0_ref[0] + p1_ref[0] + xs3_ref[...])
                     + b3_ref[...], 0.0)
    xs4_ref[...] = dinv * jnp.dot(h3, W4_ref[...],
                                  preferred_element_type=jnp.float32)


def _tc4(p0, p1, xs3, dinv, b3, W4):
    return pl.pallas_call(
        _tc4_body,
        grid=(_GRID,),
        in_specs=[
            pl.BlockSpec((1, _BLK, 64), lambda i: (0, i, 0)),
            pl.BlockSpec((1, _BLK, 64), lambda i: (1, i, 0)),
            pl.BlockSpec((_BLK, 64), lambda i: (i, 0)),
            pl.BlockSpec((_BLK, 1), lambda i: (i, 0)),
            pl.BlockSpec((1, 64), lambda i: (0, 0)),
            pl.BlockSpec((64, 32), lambda i: (0, 0)),
        ],
        out_specs=pl.BlockSpec((_BLK, 32), lambda i: (i, 0)),
        out_shape=jax.ShapeDtypeStruct((N, 32), jnp.float32),
    )(p0, p1, xs3, dinv, b3, W4)


def _tc5_body(p0_ref, p1_ref, xs4_ref, dinv_ref, b4_ref, batch_ref,
              W_ihT_ref, bg_ref, W_fc_ref, b_fc_ref, out_ref, ssum, cnt):
    i = pl.program_id(0)

    @pl.when(i == 0)
    def _():
        ssum[...] = jnp.zeros_like(ssum)
        cnt[...] = jnp.zeros_like(cnt)

    h4 = jnp.maximum(
        dinv_ref[...] * (p0_ref[0] + p1_ref[0] + xs4_ref[...])
        + b4_ref[...], 0.0)
    gids = batch_ref[...][:, 0]
    onehot = (gids[None, :] ==
              lax.broadcasted_iota(jnp.int32, (NUM_GRAPHS, _BLK), 0)
              ).astype(jnp.float32)
    ssum[...] += jnp.dot(onehot, h4, preferred_element_type=jnp.float32)
    cnt[...] += jnp.sum(onehot, axis=1, keepdims=True)

    @pl.when(i == _GRID - 1)
    def _():
        emb = ssum[...] / jnp.maximum(cnt[...], 1.0)
        gates = jnp.dot(emb, W_ihT_ref[...],
                        preferred_element_type=jnp.float32) + bg_ref[...]
        i_g = gates[:, 0 * LSTM_H:1 * LSTM_H]
        g_g = gates[:, 2 * LSTM_H:3 * LSTM_H]
        o_g = gates[:, 3 * LSTM_H:4 * LSTM_H]
        c1 = jax.nn.sigmoid(i_g) * jnp.tanh(g_g)
        h1 = jax.nn.sigmoid(o_g) * jnp.tanh(c1)
        out_ref[...] = jnp.dot(h1, W_fc_ref[...],
                               preferred_element_type=jnp.float32) + b_fc_ref[...]


def _tc5(p0, p1, xs4, dinv, b4, batch2d, W_ihT, bg, W_fc, b_fc):
    return pl.pallas_call(
        _tc5_body,
        grid=(_GRID,),
        in_specs=[
            pl.BlockSpec((1, _BLK, 32), lambda i: (0, i, 0)),
            pl.BlockSpec((1, _BLK, 32), lambda i: (1, i, 0)),
            pl.BlockSpec((_BLK, 32), lambda i: (i, 0)),
            pl.BlockSpec((_BLK, 1), lambda i: (i, 0)),
            pl.BlockSpec((1, 32), lambda i: (0, 0)),
            pl.BlockSpec((_BLK, 1), lambda i: (i, 0)),
            pl.BlockSpec((32, 4 * LSTM_H), lambda i: (0, 0)),
            pl.BlockSpec((1, 4 * LSTM_H), lambda i: (0, 0)),
            pl.BlockSpec((LSTM_H, 8), lambda i: (0, 0)),
            pl.BlockSpec((1, 8), lambda i: (0, 0)),
        ],
        out_specs=pl.BlockSpec((NUM_GRAPHS, 8), lambda i: (0, 0)),
        out_shape=jax.ShapeDtypeStruct((NUM_GRAPHS, 8), jnp.float32),
        scratch_shapes=[
            pltpu.VMEM((NUM_GRAPHS, 32), jnp.float32),
            pltpu.VMEM((NUM_GRAPHS, 1), jnp.float32),
        ],
    )(p0, p1, xs4, dinv, b4, batch2d, W_ihT, bg, W_fc, b_fc)


def kernel(x, edge_index, batch, W1, b1, W2, b2, W3, b3, W4, b4,
           W_ih, W_hh, b_ih, b_hh, W_fc, b_fc):
    src = edge_index[0].astype(jnp.int32)
    dst = edge_index[1].astype(jnp.int32)

    npad = EP - E
    fill = jnp.arange(npad, dtype=jnp.int32)
    src_p = jnp.concatenate([src, fill % 128])
    dst_p = jnp.concatenate([dst, TRASH + (fill % 16)])
    dummy = jnp.arange(CHUNK, dtype=jnp.int32)

    esrc = jnp.concatenate(
        [src_p.reshape(NW, C_E, CHUNK),
         jnp.broadcast_to(dummy, (NW, 2, CHUNK))], axis=1)
    edst = dst_p.reshape(NW, C_E, CHUNK)
    fsrc = jnp.concatenate(
        [src_p.reshape(NS, C_F, CHUNK),
         jnp.broadcast_to(dummy, (NS, 2, CHUNK))], axis=1)
    fsrc2 = jnp.stack([fsrc, fsrc + N])
    fdst = dst_p.reshape(NS, C_F, CHUNK)

    degs = _make_degree()(edst)
    d0 = degs[:N, None]
    d1 = degs[NR:NR + N, None]

    dinv, xs1p = _tc1(d0, d1, x)

    propf = _make_propagate(64, True)
    prop3 = _make_propagate(64, False)
    prop4 = _make_propagate(32, False)

    p = propf(xs1p.reshape(2 * N, 64), fsrc2, fdst).reshape(2, NR, 64)
    xs2p = _tc2(p, p, xs1p[0], xs1p[1], dinv, W1, b1[None, :], W2)

    p = propf(xs2p.reshape(2 * N, 64), fsrc2, fdst).reshape(2, NR, 64)
    xs3 = _tc3(p, p, xs2p[0], xs2p[1], dinv, b2[None, :], W3)

    p = prop3(xs3, esrc, edst).reshape(2, NR, 64)
    xs4 = _tc4(p, p, xs3, dinv, b3[None, :], W4)

    p = prop4(xs4, esrc, edst).reshape(2, NR, 32)

    bg = (b_ih + b_hh)[None, :]
    out = _tc5(p, p, xs4, dinv, b4[None, :],
               batch.astype(jnp.int32)[:, None], W_ih.T, bg, W_fc, b_fc[None, :])
    return out


# speedup vs baseline: 30.5178x; 1.0662x over previous
import jax
import jax.numpy as jnp
from jax import lax
from jax.experimental import pallas as pl
from jax.experimental.pallas import tpu as pltpu
from jax.experimental.pallas import tpu_sc as plsc

N = 10000
E = 320000
NUM_GRAPHS = 16
LSTM_H = 128

NC = 2
NS = 16
NW = NC * NS

CHUNK = 128
C_E = 80
C_F = 160
G = NW * C_E
EP = G * CHUNK
NR = 10240
RPT = NR // NS
TRASH = N

_SC_PARAMS = pltpu.CompilerParams(use_tc_tiling_on_sc=False)


def _sc_mesh():
    return plsc.VectorSubcoreMesh(core_axis_name="c", subcore_axis_name="s")


def _make_propagate(F, feature_split):
    C = C_F if feature_split else C_E

    def body(xs_hbm, src_hbm, dst_hbm, out_hbm, src_v, dst_v,
             rows0, rows1, rows2, rows3, acc,
             g0, g1, g2, g3, s0, s1, s2, s3):
        cid = lax.axis_index("c")
        sid = lax.axis_index("s")
        rows = (rows0, rows1, rows2, rows3)
        gsem = (g0, g1, g2, g3)
        ssem = (s0, s1, s2, s3)

        def zfill(i, _):
            for jj in range(F // 16):
                rows0[i, pl.ds(jj * 16, 16)] = jnp.zeros((16,), jnp.float32)
            return 0

        lax.fori_loop(0, CHUNK, zfill, 0, unroll=False)
        r0 = sid * RPT
        for z in range(RPT // CHUNK):
            pltpu.sync_copy(rows0, acc.at[pl.ds(r0 + z * CHUNK, CHUNK)])

        if feature_split:
            base = sid * C
            pltpu.sync_copy(src_hbm.at[cid, pl.ds(base, C + 2)], src_v)
        else:
            base = (cid * NS + sid) * C
            pltpu.sync_copy(src_hbm.at[pl.ds(base, C + 2)], src_v)
        pltpu.sync_copy(dst_hbm.at[pl.ds(base, C)], dst_v)

        pltpu.async_copy(xs_hbm.at[src_v.at[0]], rows0, g0)
        pltpu.async_copy(xs_hbm.at[src_v.at[1]], rows1, g1)
        plsc.subcore_barrier()

        def step(jj, _):
            for b in range(4):
                j = jj * 4 + b
                b2 = (b + 2) % 4
                pltpu.make_async_copy(
                    xs_hbm.at[src_v.at[j]], rows[b], gsem[b]).wait()
                pltpu.async_copy(rows[b], acc.at[dst_v.at[j]], ssem[b],
                                 add=True)

                @pl.when(j >= 2)
                def _():
                    pltpu.make_async_copy(
                        rows[b2], acc.at[dst_v.at[0]], ssem[b2]).wait()

                pltpu.async_copy(xs_hbm.at[src_v.at[j + 2]], rows[b2],
                                 gsem[b2])
            return 0

        lax.fori_loop(0, C // 4, step, 0, unroll=False)
        pltpu.make_async_copy(rows[2], acc.at[dst_v.at[0]], ssem[2]).wait()
        pltpu.make_async_copy(rows[3], acc.at[dst_v.at[0]], ssem[3]).wait()
        pltpu.make_async_copy(xs_hbm.at[src_v.at[0]], rows[0], gsem[0]).wait()
        pltpu.make_async_copy(xs_hbm.at[src_v.at[1]], rows[1], gsem[1]).wait()

        plsc.subcore_barrier()
        pltpu.sync_copy(acc.at[pl.ds(sid * RPT, RPT)],
                        out_hbm.at[cid, pl.ds(sid * RPT, RPT)])

    return pl.kernel(
        body,
        out_type=jax.ShapeDtypeStruct((NC, NR, F), jnp.float32),
        mesh=_sc_mesh(),
        compiler_params=_SC_PARAMS,
        scratch_types=[
            pltpu.VMEM((C + 2, CHUNK), jnp.int32),
            pltpu.VMEM((C, CHUNK), jnp.int32),
            pltpu.VMEM((CHUNK, F), jnp.float32),
            pltpu.VMEM((CHUNK, F), jnp.float32),
            pltpu.VMEM((CHUNK, F), jnp.float32),
            pltpu.VMEM((CHUNK, F), jnp.float32),
            pltpu.VMEM_SHARED((NR, F), jnp.float32),
            pltpu.SemaphoreType.DMA,
            pltpu.SemaphoreType.DMA,
            pltpu.SemaphoreType.DMA,
            pltpu.SemaphoreType.DMA,
            pltpu.SemaphoreType.DMA,
            pltpu.SemaphoreType.DMA,
            pltpu.SemaphoreType.DMA,
            pltpu.SemaphoreType.DMA,
        ],
    )


def _make_degree():
    def body(dst_hbm, out_hbm, dst_v, ones_v, zeros_v, acc):
        cid = lax.axis_index("c")
        sid = lax.axis_index("s")
        base = (cid * NS + sid) * C_E

        def fill(i, _):
            ones_v[pl.ds(i * 16, 16)] = jnp.full((16,), 1.0, jnp.float32)
            zeros_v[pl.ds(i * 16, 16)] = jnp.zeros((16,), jnp.float32)
            return 0

        lax.fori_loop(0, CHUNK // 16, fill, 0, unroll=False)
        r0 = sid * RPT
        for z in range(RPT // CHUNK):
            pltpu.sync_copy(zeros_v, acc.at[pl.ds(r0 + z * CHUNK, CHUNK)])
        pltpu.sync_copy(dst_hbm.at[pl.ds(base, C_E)], dst_v)
        plsc.subcore_barrier()

        def step(j, _):
            pltpu.sync_copy(ones_v, acc.at[dst_v.at[j]], add=True)
            return 0

        lax.fori_loop(0, C_E, step, 0, unroll=False)
        plsc.subcore_barrier()
        pltpu.sync_copy(acc.at[pl.ds(sid * RPT, RPT)],
                        out_hbm.at[cid, pl.ds(sid * RPT, RPT)])

    return pl.kernel(
        body,
        out_type=jax.ShapeDtypeStruct((NC, NR), jnp.float32),
        mesh=_sc_mesh(),
        compiler_params=_SC_PARAMS,
        scratch_types=[
            pltpu.VMEM((C_E, CHUNK), jnp.int32),
            pltpu.VMEM((CHUNK,), jnp.float32),
            pltpu.VMEM((CHUNK,), jnp.float32),
            pltpu.VMEM_SHARED((NR,), jnp.float32),
        ],
    )


_BLK = 2000
_GRID = N // _BLK

_ROW = lambda i: (i, 0)
_FIX = lambda i: (0, 0)


def _tc1_body(d0_ref, d1_ref, x_ref, dinv_ref, xs1_ref):
    deg = d0_ref[0] + d1_ref[0] + 1.0
    dinv = lax.rsqrt(deg)
    dinv_ref[...] = dinv
    xs1_ref[...] = dinv * x_ref[...]


def _tc1(degs, x):
    return pl.pallas_call(
        _tc1_body,
        grid=(_GRID,),
        in_specs=[
            pl.BlockSpec((1, _BLK, 1), lambda i: (0, i, 0)),
            pl.BlockSpec((1, _BLK, 1), lambda i: (1, i, 0)),
            pl.BlockSpec((_BLK, 128), _ROW),
        ],
        out_specs=[
            pl.BlockSpec((_BLK, 1), _ROW),
            pl.BlockSpec((_BLK, 128), _ROW),
        ],
        out_shape=[
            jax.ShapeDtypeStruct((N, 1), jnp.float32),
            jax.ShapeDtypeStruct((N, 128), jnp.float32),
        ],
    )(degs, degs, x)


def _tc2_body(p0_ref, p1_ref, xs1_ref, dinv_ref, W1_ref, b1_ref, W2_ref,
              xs2_ref):
    dinv = dinv_ref[...]
    y1 = dinv * (jnp.concatenate([p0_ref[0], p1_ref[0]], axis=1)
                 + xs1_ref[...])
    h1 = jnp.maximum(
        jnp.dot(y1, W1_ref[...], preferred_element_type=jnp.float32)
        + b1_ref[...], 0.0)
    xs2_ref[...] = dinv * jnp.dot(h1, W2_ref[...],
                                  preferred_element_type=jnp.float32)


def _tc2(p, xs1, dinv, W1, b1, W2):
    return pl.pallas_call(
        _tc2_body,
        grid=(_GRID,),
        in_specs=[
            pl.BlockSpec((1, _BLK, 64), lambda i: (0, i, 0)),
            pl.BlockSpec((1, _BLK, 64), lambda i: (1, i, 0)),
            pl.BlockSpec((_BLK, 128), _ROW),
            pl.BlockSpec((_BLK, 1), _ROW),
            pl.BlockSpec((128, 256), _FIX),
            pl.BlockSpec((1, 256), _FIX),
            pl.BlockSpec((256, 128), _FIX),
        ],
        out_specs=pl.BlockSpec((_BLK, 128), _ROW),
        out_shape=jax.ShapeDtypeStruct((N, 128), jnp.float32),
    )(p, p, xs1, dinv, W1, b1, W2)


def _tc3_body(p0_ref, p1_ref, xs2_ref, dinv_ref, b2_ref, W3_ref, xs3_ref):
    dinv = dinv_ref[...]
    h2 = jnp.maximum(
        dinv * (jnp.concatenate([p0_ref[0], p1_ref[0]], axis=1)
                + xs2_ref[...]) + b2_ref[...], 0.0)
    xs3_ref[...] = dinv * jnp.dot(h2, W3_ref[...],
                                  preferred_element_type=jnp.float32)


def _tc3(p, xs2, dinv, b2, W3):
    return pl.pallas_call(
        _tc3_body,
        grid=(_GRID,),
        in_specs=[
            pl.BlockSpec((1, _BLK, 64), lambda i: (0, i, 0)),
            pl.BlockSpec((1, _BLK, 64), lambda i: (1, i, 0)),
            pl.BlockSpec((_BLK, 128), _ROW),
            pl.BlockSpec((_BLK, 1), _ROW),
            pl.BlockSpec((1, 128), _FIX),
            pl.BlockSpec((128, 64), _FIX),
        ],
        out_specs=pl.BlockSpec((_BLK, 64), _ROW),
        out_shape=jax.ShapeDtypeStruct((N, 64), jnp.float32),
    )(p, p, xs2, dinv, b2, W3)


def _tc4_body(p0_ref, p1_ref, xs3_ref, dinv_ref, b3_ref, W4_ref, xs4_ref):
    dinv = dinv_ref[...]
    h3 = jnp.maximum(dinv * (p0_ref[0] + p1_ref[0] + xs3_ref[...])
                     + b3_ref[...], 0.0)
    xs4_ref[...] = dinv * jnp.dot(h3, W4_ref[...],
                                  preferred_element_type=jnp.float32)


def _tc4(p, xs3, dinv, b3, W4):
    return pl.pallas_call(
        _tc4_body,
        grid=(_GRID,),
        in_specs=[
            pl.BlockSpec((1, _BLK, 64), lambda i: (0, i, 0)),
            pl.BlockSpec((1, _BLK, 64), lambda i: (1, i, 0)),
            pl.BlockSpec((_BLK, 64), _ROW),
            pl.BlockSpec((_BLK, 1), _ROW),
            pl.BlockSpec((1, 64), _FIX),
            pl.BlockSpec((64, 32), _FIX),
        ],
        out_specs=pl.BlockSpec((_BLK, 32), _ROW),
        out_shape=jax.ShapeDtypeStruct((N, 32), jnp.float32),
    )(p, p, xs3, dinv, b3, W4)


def _tc5_body(p0_ref, p1_ref, xs4_ref, dinv_ref, b4_ref, batch_ref,
              W_ihT_ref, bg_ref, W_fc_ref, b_fc_ref, out_ref, ssum, cnt):
    i = pl.program_id(0)

    @pl.when(i == 0)
    def _():
        ssum[...] = jnp.zeros_like(ssum)
        cnt[...] = jnp.zeros_like(cnt)

    h4 = jnp.maximum(
        dinv_ref[...] * (p0_ref[0] + p1_ref[0] + xs4_ref[...])
        + b4_ref[...], 0.0)
    gids = batch_ref[...][:, 0]
    onehot = (gids[None, :] ==
              lax.broadcasted_iota(jnp.int32, (NUM_GRAPHS, _BLK), 0)
              ).astype(jnp.float32)
    ssum[...] += jnp.dot(onehot, h4, preferred_element_type=jnp.float32)
    cnt[...] += jnp.sum(onehot, axis=1, keepdims=True)

    @pl.when(i == _GRID - 1)
    def _():
        emb = ssum[...] / jnp.maximum(cnt[...], 1.0)
        gates = jnp.dot(emb, W_ihT_ref[...],
                        preferred_element_type=jnp.float32) + bg_ref[...]
        i_g = gates[:, 0 * LSTM_H:1 * LSTM_H]
        g_g = gates[:, 2 * LSTM_H:3 * LSTM_H]
        o_g = gates[:, 3 * LSTM_H:4 * LSTM_H]
        c1 = jax.nn.sigmoid(i_g) * jnp.tanh(g_g)
        h1 = jax.nn.sigmoid(o_g) * jnp.tanh(c1)
        out_ref[...] = jnp.dot(h1, W_fc_ref[...],
                               preferred_element_type=jnp.float32) + b_fc_ref[...]


def _tc5(p, xs4, dinv, b4, batch2d, W_ihT, bg, W_fc, b_fc):
    return pl.pallas_call(
        _tc5_body,
        grid=(_GRID,),
        in_specs=[
            pl.BlockSpec((1, _BLK, 32), lambda i: (0, i, 0)),
            pl.BlockSpec((1, _BLK, 32), lambda i: (1, i, 0)),
            pl.BlockSpec((_BLK, 32), _ROW),
            pl.BlockSpec((_BLK, 1), _ROW),
            pl.BlockSpec((1, 32), _FIX),
            pl.BlockSpec((_BLK, 1), _ROW),
            pl.BlockSpec((32, 4 * LSTM_H), _FIX),
            pl.BlockSpec((1, 4 * LSTM_H), _FIX),
            pl.BlockSpec((LSTM_H, 8), _FIX),
            pl.BlockSpec((1, 8), _FIX),
        ],
        out_specs=pl.BlockSpec((NUM_GRAPHS, 8), _FIX),
        out_shape=jax.ShapeDtypeStruct((NUM_GRAPHS, 8), jnp.float32),
        scratch_shapes=[
            pltpu.VMEM((NUM_GRAPHS, 32), jnp.float32),
            pltpu.VMEM((NUM_GRAPHS, 1), jnp.float32),
        ],
    )(p, p, xs4, dinv, b4, batch2d, W_ihT, bg, W_fc, b_fc)


def kernel(x, edge_index, batch, W1, b1, W2, b2, W3, b3, W4, b4,
           W_ih, W_hh, b_ih, b_hh, W_fc, b_fc):
    src = edge_index[0].astype(jnp.int32)
    dst = edge_index[1].astype(jnp.int32)

    npad = EP - E
    fill = jnp.arange(npad + 2 * CHUNK, dtype=jnp.int32)
    src_e = jnp.concatenate([src, fill % 128]).reshape(G + 2, CHUNK)
    dst_f = jnp.concatenate([dst, TRASH + (fill[:npad] % 16)]
                            ).reshape(G, CHUNK)
    src_f = jnp.stack([2 * src_e, 2 * src_e + 1])

    degs = _make_degree()(dst_f)
    dinv, xs1 = _tc1(degs[..., None], x)

    propf = _make_propagate(64, True)
    prop3 = _make_propagate(64, False)
    prop4 = _make_propagate(32, False)

    p = propf(xs1.reshape(2 * N, 64), src_f, dst_f)
    xs2 = _tc2(p, xs1, dinv, W1, b1[None, :], W2)

    p = propf(xs2.reshape(2 * N, 64), src_f, dst_f)
    xs3 = _tc3(p, xs2, dinv, b2[None, :], W3)

    p = prop3(xs3, src_e, dst_f)
    xs4 = _tc4(p, xs3, dinv, b3[None, :], W4)

    p = prop4(xs4, src_e, dst_f)

    bg = (b_ih + b_hh)[None, :]
    out = _tc5(p, xs4, dinv, b4[None, :],
               batch.astype(jnp.int32)[:, None], W_ih.T, bg, W_fc,
               b_fc[None, :])
    return out


# speedup vs baseline: 37.6289x; 1.2330x over previous
import jax
import jax.numpy as jnp
from jax import lax
from jax.experimental import pallas as pl
from jax.experimental.pallas import tpu as pltpu
from jax.experimental.pallas import tpu_sc as plsc

N = 10000
E = 320000
NUM_GRAPHS = 16
LSTM_H = 128

NC = 2
NS = 16
NW = NC * NS

CHUNK = 240
ZCH = 128
C_E = 43
C_F = 86
G = NW * C_E
EP = G * CHUNK
NR = 10240
RPT = NR // NS
TRASH = N

_SC_PARAMS = pltpu.CompilerParams(use_tc_tiling_on_sc=False)


def _sc_mesh():
    return plsc.VectorSubcoreMesh(core_axis_name="c", subcore_axis_name="s")


def _make_propagate(F, feature_split):
    C = C_F if feature_split else C_E

    def body(xs_hbm, src_hbm, dst_hbm, out_hbm, src_v, dst_v,
             rows0, rows1, rows2, acc, g0, g1, g2, s0, s1, s2):
        cid = lax.axis_index("c")
        sid = lax.axis_index("s")
        rows = (rows0, rows1, rows2)
        gsem = (g0, g1, g2)
        ssem = (s0, s1, s2)

        def zfill(i, _):
            for jj in range(F // 16):
                rows0[i, pl.ds(jj * 16, 16)] = jnp.zeros((16,), jnp.float32)
            return 0

        lax.fori_loop(0, ZCH, zfill, 0, unroll=False)
        r0 = sid * RPT
        for z in range(RPT // ZCH):
            pltpu.sync_copy(rows0.at[pl.ds(0, ZCH)],
                            acc.at[pl.ds(r0 + z * ZCH, ZCH)])

        if feature_split:
            base = sid * C
            pltpu.sync_copy(src_hbm.at[cid, pl.ds(base, C + 2)], src_v)
        else:
            base = (cid * NS + sid) * C
            pltpu.sync_copy(src_hbm.at[0, pl.ds(base, C + 2)], src_v)
        pltpu.sync_copy(dst_hbm.at[pl.ds(base, C)], dst_v)

        pltpu.async_copy(xs_hbm.at[src_v.at[0]], rows0, g0)
        pltpu.async_copy(xs_hbm.at[src_v.at[1]], rows1, g1)
        plsc.subcore_barrier()

        def slot(j, b):
            b2 = (b + 2) % 3
            pltpu.make_async_copy(
                xs_hbm.at[src_v.at[j]], rows[b], gsem[b]).wait()
            pltpu.async_copy(rows[b], acc.at[dst_v.at[j]], ssem[b], add=True)
            pltpu.make_async_copy(
                rows[b2], acc.at[dst_v.at[0]], ssem[b2]).wait()
            pltpu.async_copy(xs_hbm.at[src_v.at[j + 2]], rows[b2], gsem[b2])

        pltpu.make_async_copy(xs_hbm.at[src_v.at[0]], rows[0], gsem[0]).wait()
        pltpu.async_copy(rows[0], acc.at[dst_v.at[0]], ssem[0], add=True)
        pltpu.async_copy(xs_hbm.at[src_v.at[2]], rows[2], gsem[2])

        def step(jj, _):
            for b in range(3):
                slot(1 + jj * 3 + b, (1 + b) % 3)
            return 0

        lax.fori_loop(0, (C - 1) // 3, step, 0, unroll=False)
        for r in range((C - 1) % 3):
            j = C - ((C - 1) % 3) + r
            slot(j, j % 3)
        pltpu.make_async_copy(
            rows[(C - 1) % 3], acc.at[dst_v.at[0]], ssem[(C - 1) % 3]).wait()
        pltpu.make_async_copy(
            xs_hbm.at[src_v.at[0]], rows[C % 3], gsem[C % 3]).wait()
        pltpu.make_async_copy(
            xs_hbm.at[src_v.at[0]], rows[(C + 1) % 3], gsem[(C + 1) % 3]).wait()

        plsc.subcore_barrier()
        pltpu.sync_copy(acc.at[pl.ds(sid * RPT, RPT)],
                        out_hbm.at[pl.ds(sid * RPT, RPT), pl.ds(cid * 64, F)])

    return pl.kernel(
        body,
        out_type=jax.ShapeDtypeStruct((NR, 128), jnp.float32),
        mesh=_sc_mesh(),
        compiler_params=_SC_PARAMS,
        scratch_types=[
            pltpu.VMEM((C + 2, CHUNK), jnp.int32),
            pltpu.VMEM((C, CHUNK), jnp.int32),
            pltpu.VMEM((CHUNK, F), jnp.float32),
            pltpu.VMEM((CHUNK, F), jnp.float32),
            pltpu.VMEM((CHUNK, F), jnp.float32),
            pltpu.VMEM_SHARED((NR, F), jnp.float32),
            pltpu.SemaphoreType.DMA,
            pltpu.SemaphoreType.DMA,
            pltpu.SemaphoreType.DMA,
            pltpu.SemaphoreType.DMA,
            pltpu.SemaphoreType.DMA,
            pltpu.SemaphoreType.DMA,
        ],
    )


def _make_degree():
    def body(dst_hbm, out_hbm, dst_v, ones_v, zeros_v, acc):
        cid = lax.axis_index("c")
        sid = lax.axis_index("s")
        base = (cid * NS + sid) * C_E

        def fill(i, _):
            ones_v[pl.ds(i * 16, 16)] = jnp.full((16,), 1.0, jnp.float32)
            zeros_v[pl.ds(i * 16, 16)] = jnp.zeros((16,), jnp.float32)
            return 0

        lax.fori_loop(0, CHUNK // 16, fill, 0, unroll=False)
        r0 = sid * RPT
        for z in range(RPT // ZCH):
            pltpu.sync_copy(zeros_v.at[pl.ds(0, ZCH)],
                            acc.at[pl.ds(r0 + z * ZCH, ZCH)])
        pltpu.sync_copy(dst_hbm.at[pl.ds(base, C_E)], dst_v)
        plsc.subcore_barrier()

        def step(j, _):
            pltpu.sync_copy(ones_v, acc.at[dst_v.at[j]], add=True)
            return 0

        lax.fori_loop(0, C_E, step, 0, unroll=False)
        plsc.subcore_barrier()
        pltpu.sync_copy(acc.at[pl.ds(sid * RPT, RPT)],
                        out_hbm.at[cid, pl.ds(sid * RPT, RPT)])

    return pl.kernel(
        body,
        out_type=jax.ShapeDtypeStruct((NC, NR), jnp.float32),
        mesh=_sc_mesh(),
        compiler_params=_SC_PARAMS,
        scratch_types=[
            pltpu.VMEM((C_E, CHUNK), jnp.int32),
            pltpu.VMEM((CHUNK,), jnp.float32),
            pltpu.VMEM((CHUNK,), jnp.float32),
            pltpu.VMEM_SHARED((NR,), jnp.float32),
        ],
    )


_BLK = 2000
_GRID = N // _BLK

_ROW = lambda i: (i, 0)
_FIX = lambda i: (0, 0)


def _tc1_body(d0_ref, d1_ref, x_ref, dinv_ref, xs1_ref):
    deg = d0_ref[0] + d1_ref[0] + 1.0
    dinv = lax.rsqrt(deg)
    dinv_ref[...] = dinv
    xs1_ref[...] = dinv * x_ref[...]


def _tc1(degs, x):
    return pl.pallas_call(
        _tc1_body,
        grid=(_GRID,),
        in_specs=[
            pl.BlockSpec((1, _BLK, 1), lambda i: (0, i, 0)),
            pl.BlockSpec((1, _BLK, 1), lambda i: (1, i, 0)),
            pl.BlockSpec((_BLK, 128), _ROW),
        ],
        out_specs=[
            pl.BlockSpec((_BLK, 1), _ROW),
            pl.BlockSpec((_BLK, 128), _ROW),
        ],
        out_shape=[
            jax.ShapeDtypeStruct((N, 1), jnp.float32),
            jax.ShapeDtypeStruct((N, 128), jnp.float32),
        ],
    )(degs, degs, x)


def _tc2_body(p_ref, xs1_ref, dinv_ref, W1_ref, b1_ref, W2_ref,
              xs2_ref):
    dinv = dinv_ref[...]
    y1 = dinv * (p_ref[...] + xs1_ref[...])
    h1 = jnp.maximum(
        jnp.dot(y1, W1_ref[...], preferred_element_type=jnp.float32)
        + b1_ref[...], 0.0)
    xs2_ref[...] = dinv * jnp.dot(h1, W2_ref[...],
                                  preferred_element_type=jnp.float32)


def _tc2(p, xs1, dinv, W1, b1, W2):
    return pl.pallas_call(
        _tc2_body,
        grid=(_GRID,),
        in_specs=[
            pl.BlockSpec((_BLK, 128), _ROW),
            pl.BlockSpec((_BLK, 128), _ROW),
            pl.BlockSpec((_BLK, 1), _ROW),
            pl.BlockSpec((128, 256), _FIX),
            pl.BlockSpec((1, 256), _FIX),
            pl.BlockSpec((256, 128), _FIX),
        ],
        out_specs=pl.BlockSpec((_BLK, 128), _ROW),
        out_shape=jax.ShapeDtypeStruct((N, 128), jnp.float32),
    )(p, xs1, dinv, W1, b1, W2)


def _tc3_body(p_ref, xs2_ref, dinv_ref, b2_ref, W3_ref, xs3_ref):
    dinv = dinv_ref[...]
    h2 = jnp.maximum(dinv * (p_ref[...] + xs2_ref[...]) + b2_ref[...], 0.0)
    t3 = dinv * jnp.dot(h2, W3_ref[...], preferred_element_type=jnp.float32)
    xs3_ref[...] = jnp.concatenate([t3, jnp.zeros_like(t3)], axis=1)


def _tc3(p, xs2, dinv, b2, W3):
    return pl.pallas_call(
        _tc3_body,
        grid=(_GRID,),
        in_specs=[
            pl.BlockSpec((_BLK, 128), _ROW),
            pl.BlockSpec((_BLK, 128), _ROW),
            pl.BlockSpec((_BLK, 1), _ROW),
            pl.BlockSpec((1, 128), _FIX),
            pl.BlockSpec((128, 64), _FIX),
        ],
        out_specs=pl.BlockSpec((_BLK, 128), _ROW),
        out_shape=jax.ShapeDtypeStruct((N, 128), jnp.float32),
    )(p, xs2, dinv, b2, W3)


def _tc4_body(p_ref, xs3_ref, dinv_ref, b3_ref, W4_ref, xs4_ref):
    dinv = dinv_ref[...]
    pk = p_ref[...]
    h3 = jnp.maximum(dinv * (pk[:, :64] + pk[:, 64:] + xs3_ref[:, :64])
                     + b3_ref[...], 0.0)
    t4 = dinv * jnp.dot(h3, W4_ref[...], preferred_element_type=jnp.float32)
    xs4_ref[...] = jnp.concatenate(
        [t4, jnp.zeros((t4.shape[0], 96), jnp.float32)], axis=1)


def _tc4(p, xs3, dinv, b3, W4):
    return pl.pallas_call(
        _tc4_body,
        grid=(_GRID,),
        in_specs=[
            pl.BlockSpec((_BLK, 128), _ROW),
            pl.BlockSpec((_BLK, 128), _ROW),
            pl.BlockSpec((_BLK, 1), _ROW),
            pl.BlockSpec((1, 64), _FIX),
            pl.BlockSpec((64, 32), _FIX),
        ],
        out_specs=pl.BlockSpec((_BLK, 128), _ROW),
        out_shape=jax.ShapeDtypeStruct((N, 128), jnp.float32),
    )(p, xs3, dinv, b3, W4)


def _tc5_body(p_ref, xs4_ref, dinv_ref, b4_ref, batch_ref,
              W_ihT_ref, bg_ref, W_fc_ref, b_fc_ref, out_ref, ssum, cnt):
    i = pl.program_id(0)

    @pl.when(i == 0)
    def _():
        ssum[...] = jnp.zeros_like(ssum)
        cnt[...] = jnp.zeros_like(cnt)

    pk = p_ref[...]
    h4 = jnp.maximum(
        dinv_ref[...] * (pk[:, 0:32] + pk[:, 64:96] + xs4_ref[:, :32])
        + b4_ref[...], 0.0)
    gids = batch_ref[...][:, 0]
    onehot = (gids[None, :] ==
              lax.broadcasted_iota(jnp.int32, (NUM_GRAPHS, _BLK), 0)
              ).astype(jnp.float32)
    ssum[...] += jnp.dot(onehot, h4, preferred_element_type=jnp.float32)
    cnt[...] += jnp.sum(onehot, axis=1, keepdims=True)

    @pl.when(i == _GRID - 1)
    def _():
        emb = ssum[...] / jnp.maximum(cnt[...], 1.0)
        gates = jnp.dot(emb, W_ihT_ref[...],
                        preferred_element_type=jnp.float32) + bg_ref[...]
        i_g = gates[:, 0 * LSTM_H:1 * LSTM_H]
        g_g = gates[:, 2 * LSTM_H:3 * LSTM_H]
        o_g = gates[:, 3 * LSTM_H:4 * LSTM_H]
        c1 = jax.nn.sigmoid(i_g) * jnp.tanh(g_g)
        h1 = jax.nn.sigmoid(o_g) * jnp.tanh(c1)
        out_ref[...] = jnp.dot(h1, W_fc_ref[...],
                               preferred_element_type=jnp.float32) + b_fc_ref[...]


def _tc5(p, xs4, dinv, b4, batch2d, W_ihT, bg, W_fc, b_fc):
    return pl.pallas_call(
        _tc5_body,
        grid=(_GRID,),
        in_specs=[
            pl.BlockSpec((_BLK, 128), _ROW),
            pl.BlockSpec((_BLK, 128), _ROW),
            pl.BlockSpec((_BLK, 1), _ROW),
            pl.BlockSpec((1, 32), _FIX),
            pl.BlockSpec((_BLK, 1), _ROW),
            pl.BlockSpec((32, 4 * LSTM_H), _FIX),
            pl.BlockSpec((1, 4 * LSTM_H), _FIX),
            pl.BlockSpec((LSTM_H, 8), _FIX),
            pl.BlockSpec((1, 8), _FIX),
        ],
        out_specs=pl.BlockSpec((NUM_GRAPHS, 8), _FIX),
        out_shape=jax.ShapeDtypeStruct((NUM_GRAPHS, 8), jnp.float32),
        scratch_shapes=[
            pltpu.VMEM((NUM_GRAPHS, 32), jnp.float32),
            pltpu.VMEM((NUM_GRAPHS, 1), jnp.float32),
        ],
    )(p, xs4, dinv, b4, batch2d, W_ihT, bg, W_fc, b_fc)


def kernel(x, edge_index, batch, W1, b1, W2, b2, W3, b3, W4, b4,
           W_ih, W_hh, b_ih, b_hh, W_fc, b_fc):
    src = edge_index[0].astype(jnp.int32)
    dst = edge_index[1].astype(jnp.int32)

    npad = EP - E
    fill = jnp.arange(npad + 2 * CHUNK, dtype=jnp.int32)
    src_e = jnp.concatenate([src, fill % 128]).reshape(G + 2, CHUNK)
    dst_f = jnp.concatenate([dst, TRASH + (fill[:npad] % 16)]
                            ).reshape(G, CHUNK)
    src_f = jnp.stack([2 * src_e, 2 * src_e + 1])
    src_q = 4 * src_e[None]

    degs = _make_degree()(dst_f)
    dinv, xs1 = _tc1(degs[..., None], x)

    propf = _make_propagate(64, True)
    prop3 = _make_propagate(64, False)
    prop4 = _make_propagate(32, False)

    p = propf(xs1.reshape(2 * N, 64), src_f, dst_f)
    xs2 = _tc2(p, xs1, dinv, W1, b1[None, :], W2)

    p = propf(xs2.reshape(2 * N, 64), src_f, dst_f)
    xs3 = _tc3(p, xs2, dinv, b2[None, :], W3)

    p = prop3(xs3.reshape(2 * N, 64), src_f, dst_f)
    xs4 = _tc4(p, xs3, dinv, b3[None, :], W4)

    p = prop4(xs4.reshape(4 * N, 32), src_q, dst_f)

    bg = (b_ih + b_hh)[None, :]
    out = _tc5(p, xs4, dinv, b4[None, :],
               batch.astype(jnp.int32)[:, None], W_ih.T, bg, W_fc,
               b_fc[None, :])
    return out


# speedup vs baseline: 38.2454x; 1.0164x over previous
import jax
import jax.numpy as jnp
from jax import lax
from jax.experimental import pallas as pl
from jax.experimental.pallas import tpu as pltpu
from jax.experimental.pallas import tpu_sc as plsc

N = 10000
E = 320000
NUM_GRAPHS = 16
LSTM_H = 128

NC = 2
NS = 16
NW = NC * NS

CHUNK = 224
ZCH = 128
C_E = 45
C_F = 90
G = NW * C_E
EP = G * CHUNK
NR = 10240
RPT = NR // NS
TRASH = N

_SC_PARAMS = pltpu.CompilerParams(use_tc_tiling_on_sc=False)


def _sc_mesh():
    return plsc.VectorSubcoreMesh(core_axis_name="c", subcore_axis_name="s")


def _make_propagate(F, feature_split):
    C = C_F if feature_split else C_E

    def body(xs_hbm, src_hbm, dst_hbm, out_hbm, src_v, dst_v,
             rows0, rows1, rows2, acc, g0, g1, g2, s0, s1, s2):
        cid = lax.axis_index("c")
        sid = lax.axis_index("s")
        rows = (rows0, rows1, rows2)
        gsem = (g0, g1, g2)
        ssem = (s0, s1, s2)

        def zfill(i, _):
            for jj in range(F // 16):
                rows0[i, pl.ds(jj * 16, 16)] = jnp.zeros((16,), jnp.float32)
            return 0

        lax.fori_loop(0, ZCH, zfill, 0, unroll=False)
        r0 = sid * RPT
        for z in range(RPT // ZCH):
            pltpu.sync_copy(rows0.at[pl.ds(0, ZCH)],
                            acc.at[pl.ds(r0 + z * ZCH, ZCH)])

        if feature_split:
            base = sid * C
            pltpu.sync_copy(src_hbm.at[cid, pl.ds(base, C + 2)], src_v)
        else:
            base = (cid * NS + sid) * C
            pltpu.sync_copy(src_hbm.at[0, pl.ds(base, C + 2)], src_v)
        pltpu.sync_copy(dst_hbm.at[pl.ds(base, C)], dst_v)

        pltpu.async_copy(xs_hbm.at[src_v.at[0]], rows0, g0)
        pltpu.async_copy(xs_hbm.at[src_v.at[1]], rows1, g1)
        plsc.subcore_barrier()

        def slot(j, b):
            b2 = (b + 2) % 3
            pltpu.make_async_copy(
                xs_hbm.at[src_v.at[j]], rows[b], gsem[b]).wait()
            pltpu.async_copy(rows[b], acc.at[dst_v.at[j]], ssem[b], add=True)
            pltpu.make_async_copy(
                rows[b2], acc.at[dst_v.at[0]], ssem[b2]).wait()
            pltpu.async_copy(xs_hbm.at[src_v.at[j + 2]], rows[b2], gsem[b2])

        pltpu.make_async_copy(xs_hbm.at[src_v.at[0]], rows[0], gsem[0]).wait()
        pltpu.async_copy(rows[0], acc.at[dst_v.at[0]], ssem[0], add=True)
        pltpu.async_copy(xs_hbm.at[src_v.at[2]], rows[2], gsem[2])

        def step(jj, _):
            for b in range(3):
                slot(1 + jj * 3 + b, (1 + b) % 3)
            return 0

        lax.fori_loop(0, (C - 1) // 3, step, 0, unroll=False)
        for r in range((C - 1) % 3):
            j = C - ((C - 1) % 3) + r
            slot(j, j % 3)
        pltpu.make_async_copy(
            rows[(C - 1) % 3], acc.at[dst_v.at[0]], ssem[(C - 1) % 3]).wait()
        pltpu.make_async_copy(
            xs_hbm.at[src_v.at[0]], rows[C % 3], gsem[C % 3]).wait()
        pltpu.make_async_copy(
            xs_hbm.at[src_v.at[0]], rows[(C + 1) % 3], gsem[(C + 1) % 3]).wait()

        plsc.subcore_barrier()
        pltpu.sync_copy(acc.at[pl.ds(sid * RPT, RPT)],
                        out_hbm.at[pl.ds(sid * RPT, RPT), pl.ds(cid * 64, F)])

    return pl.kernel(
        body,
        out_type=jax.ShapeDtypeStruct((NR, 128), jnp.float32),
        mesh=_sc_mesh(),
        compiler_params=_SC_PARAMS,
        scratch_types=[
            pltpu.VMEM((C + 2, CHUNK), jnp.int32),
            pltpu.VMEM((C, CHUNK), jnp.int32),
            pltpu.VMEM((CHUNK, F), jnp.float32),
            pltpu.VMEM((CHUNK, F), jnp.float32),
            pltpu.VMEM((CHUNK, F), jnp.float32),
            pltpu.VMEM_SHARED((NR, F), jnp.float32),
            pltpu.SemaphoreType.DMA,
            pltpu.SemaphoreType.DMA,
            pltpu.SemaphoreType.DMA,
            pltpu.SemaphoreType.DMA,
            pltpu.SemaphoreType.DMA,
            pltpu.SemaphoreType.DMA,
        ],
    )


def _make_degree():
    def body(dst_hbm, out_hbm, dst_v, ones_v, zeros_v, acc):
        cid = lax.axis_index("c")
        sid = lax.axis_index("s")
        base = (cid * NS + sid) * C_E

        def fill(i, _):
            ones_v[pl.ds(i * 16, 16)] = jnp.full((16,), 1.0, jnp.float32)
            zeros_v[pl.ds(i * 16, 16)] = jnp.zeros((16,), jnp.float32)
            return 0

        lax.fori_loop(0, CHUNK // 16, fill, 0, unroll=False)
        r0 = sid * RPT
        for z in range(RPT // ZCH):
            pltpu.sync_copy(zeros_v.at[pl.ds(0, ZCH)],
                            acc.at[pl.ds(r0 + z * ZCH, ZCH)])
        pltpu.sync_copy(dst_hbm.at[pl.ds(base, C_E)], dst_v)
        plsc.subcore_barrier()

        def step(j, _):
            pltpu.sync_copy(ones_v, acc.at[dst_v.at[j]], add=True)
            return 0

        lax.fori_loop(0, C_E, step, 0, unroll=False)
        plsc.subcore_barrier()
        pltpu.sync_copy(acc.at[pl.ds(sid * RPT, RPT)],
                        out_hbm.at[cid, pl.ds(sid * RPT, RPT)])

    return pl.kernel(
        body,
        out_type=jax.ShapeDtypeStruct((NC, NR), jnp.float32),
        mesh=_sc_mesh(),
        compiler_params=_SC_PARAMS,
        scratch_types=[
            pltpu.VMEM((C_E, CHUNK), jnp.int32),
            pltpu.VMEM((CHUNK,), jnp.float32),
            pltpu.VMEM((CHUNK,), jnp.float32),
            pltpu.VMEM_SHARED((NR,), jnp.float32),
        ],
    )


_BLK = 2000
_GRID = N // _BLK

_ROW = lambda i: (i, 0)
_FIX = lambda i: (0, 0)


def _tc1_body(d0_ref, d1_ref, x_ref, dinv_ref, xs1_ref):
    deg = d0_ref[0] + d1_ref[0] + 1.0
    dinv = lax.rsqrt(deg)
    dinv_ref[...] = dinv
    xs1_ref[...] = dinv * x_ref[...]


def _tc1(degs, x):
    return pl.pallas_call(
        _tc1_body,
        grid=(_GRID,),
        in_specs=[
            pl.BlockSpec((1, _BLK, 1), lambda i: (0, i, 0)),
            pl.BlockSpec((1, _BLK, 1), lambda i: (1, i, 0)),
            pl.BlockSpec((_BLK, 128), _ROW),
        ],
        out_specs=[
            pl.BlockSpec((_BLK, 1), _ROW),
            pl.BlockSpec((_BLK, 128), _ROW),
        ],
        out_shape=[
            jax.ShapeDtypeStruct((N, 1), jnp.float32),
            jax.ShapeDtypeStruct((N, 128), jnp.float32),
        ],
    )(degs, degs, x)


def _tc2_body(p_ref, xs1_ref, dinv_ref, W1_ref, b1_ref, W2_ref,
              xs2_ref):
    dinv = dinv_ref[...]
    y1 = dinv * (p_ref[...] + xs1_ref[...])
    h1 = jnp.maximum(
        jnp.dot(y1, W1_ref[...], preferred_element_type=jnp.float32)
        + b1_ref[...], 0.0)
    xs2_ref[...] = dinv * jnp.dot(h1, W2_ref[...],
                                  preferred_element_type=jnp.float32)


def _tc2(p, xs1, dinv, W1, b1, W2):
    return pl.pallas_call(
        _tc2_body,
        grid=(_GRID,),
        in_specs=[
            pl.BlockSpec((_BLK, 128), _ROW),
            pl.BlockSpec((_BLK, 128), _ROW),
            pl.BlockSpec((_BLK, 1), _ROW),
            pl.BlockSpec((128, 256), _FIX),
            pl.BlockSpec((1, 256), _FIX),
            pl.BlockSpec((256, 128), _FIX),
        ],
        out_specs=pl.BlockSpec((_BLK, 128), _ROW),
        out_shape=jax.ShapeDtypeStruct((N, 128), jnp.float32),
    )(p, xs1, dinv, W1, b1, W2)


def _tc3_body(p_ref, xs2_ref, dinv_ref, b2_ref, W3_ref, xs3_ref):
    dinv = dinv_ref[...]
    h2 = jnp.maximum(dinv * (p_ref[...] + xs2_ref[...]) + b2_ref[...], 0.0)
    t3 = dinv * jnp.dot(h2, W3_ref[...], preferred_element_type=jnp.float32)
    xs3_ref[...] = jnp.concatenate([t3, jnp.zeros_like(t3)], axis=1)


def _tc3(p, xs2, dinv, b2, W3):
    return pl.pallas_call(
        _tc3_body,
        grid=(_GRID,),
        in_specs=[
            pl.BlockSpec((_BLK, 128), _ROW),
            pl.BlockSpec((_BLK, 128), _ROW),
            pl.BlockSpec((_BLK, 1), _ROW),
            pl.BlockSpec((1, 128), _FIX),
            pl.BlockSpec((128, 64), _FIX),
        ],
        out_specs=pl.BlockSpec((_BLK, 128), _ROW),
        out_shape=jax.ShapeDtypeStruct((N, 128), jnp.float32),
    )(p, xs2, dinv, b2, W3)


def _tc4_body(p_ref, xs3_ref, dinv_ref, b3_ref, W4_ref, xs4_ref):
    dinv = dinv_ref[...]
    pk = p_ref[...]
    h3 = jnp.maximum(dinv * (pk[:, :64] + pk[:, 64:] + xs3_ref[:, :64])
                     + b3_ref[...], 0.0)
    t4 = dinv * jnp.dot(h3, W4_ref[...], preferred_element_type=jnp.float32)
    xs4_ref[...] = jnp.concatenate(
        [t4, jnp.zeros((t4.shape[0], 96), jnp.float32)], axis=1)


def _tc4(p, xs3, dinv, b3, W4):
    return pl.pallas_call(
        _tc4_body,
        grid=(_GRID,),
        in_specs=[
            pl.BlockSpec((_BLK, 128), _ROW),
            pl.BlockSpec((_BLK, 128), _ROW),
            pl.BlockSpec((_BLK, 1), _ROW),
            pl.BlockSpec((1, 64), _FIX),
            pl.BlockSpec((64, 32), _FIX),
        ],
        out_specs=pl.BlockSpec((_BLK, 128), _ROW),
        out_shape=jax.ShapeDtypeStruct((N, 128), jnp.float32),
    )(p, xs3, dinv, b3, W4)


def _tc5_body(p_ref, xs4_ref, dinv_ref, b4_ref, batch_ref,
              W_ihT_ref, bg_ref, W_fc_ref, b_fc_ref, out_ref, ssum, cnt):
    i = pl.program_id(0)

    @pl.when(i == 0)
    def _():
        ssum[...] = jnp.zeros_like(ssum)
        cnt[...] = jnp.zeros_like(cnt)

    pk = p_ref[...]
    h4 = jnp.maximum(
        dinv_ref[...] * (pk[:, 0:32] + pk[:, 64:96] + xs4_ref[:, :32])
        + b4_ref[...], 0.0)
    gids = batch_ref[...][:, 0]
    onehot = (gids[None, :] ==
              lax.broadcasted_iota(jnp.int32, (NUM_GRAPHS, _BLK), 0)
              ).astype(jnp.float32)
    ssum[...] += jnp.dot(onehot, h4, preferred_element_type=jnp.float32)
    cnt[...] += jnp.sum(onehot, axis=1, keepdims=True)

    @pl.when(i == _GRID - 1)
    def _():
        emb = ssum[...] / jnp.maximum(cnt[...], 1.0)
        gates = jnp.dot(emb, W_ihT_ref[...],
                        preferred_element_type=jnp.float32) + bg_ref[...]
        i_g = gates[:, 0 * LSTM_H:1 * LSTM_H]
        g_g = gates[:, 2 * LSTM_H:3 * LSTM_H]
        o_g = gates[:, 3 * LSTM_H:4 * LSTM_H]
        c1 = jax.nn.sigmoid(i_g) * jnp.tanh(g_g)
        h1 = jax.nn.sigmoid(o_g) * jnp.tanh(c1)
        out_ref[...] = jnp.dot(h1, W_fc_ref[...],
                               preferred_element_type=jnp.float32) + b_fc_ref[...]


def _tc5(p, xs4, dinv, b4, batch2d, W_ihT, bg, W_fc, b_fc):
    return pl.pallas_call(
        _tc5_body,
        grid=(_GRID,),
        in_specs=[
            pl.BlockSpec((_BLK, 128), _ROW),
            pl.BlockSpec((_BLK, 128), _ROW),
            pl.BlockSpec((_BLK, 1), _ROW),
            pl.BlockSpec((1, 32), _FIX),
            pl.BlockSpec((_BLK, 1), _ROW),
            pl.BlockSpec((32, 4 * LSTM_H), _FIX),
            pl.BlockSpec((1, 4 * LSTM_H), _FIX),
            pl.BlockSpec((LSTM_H, 8), _FIX),
            pl.BlockSpec((1, 8), _FIX),
        ],
        out_specs=pl.BlockSpec((NUM_GRAPHS, 8), _FIX),
        out_shape=jax.ShapeDtypeStruct((NUM_GRAPHS, 8), jnp.float32),
        scratch_shapes=[
            pltpu.VMEM((NUM_GRAPHS, 32), jnp.float32),
            pltpu.VMEM((NUM_GRAPHS, 1), jnp.float32),
        ],
    )(p, xs4, dinv, b4, batch2d, W_ihT, bg, W_fc, b_fc)


def kernel(x, edge_index, batch, W1, b1, W2, b2, W3, b3, W4, b4,
           W_ih, W_hh, b_ih, b_hh, W_fc, b_fc):
    src = edge_index[0].astype(jnp.int32)
    dst = edge_index[1].astype(jnp.int32)

    npad = EP - E
    fill = jnp.arange(npad + 2 * CHUNK, dtype=jnp.int32)
    src_e = jnp.concatenate([src, fill % 128]).reshape(G + 2, CHUNK)
    dst_f = jnp.concatenate([dst, TRASH + (fill[:npad] % 16)]
                            ).reshape(G, CHUNK)
    src_f = jnp.stack([2 * src_e, 2 * src_e + 1])
    src_q = 4 * src_e[None]

    degs = _make_degree()(dst_f)
    dinv, xs1 = _tc1(degs[..., None], x)

    propf = _make_propagate(64, True)
    prop3 = _make_propagate(64, False)
    prop4 = _make_propagate(32, False)

    p = propf(xs1.reshape(2 * N, 64), src_f, dst_f)
    xs2 = _tc2(p, xs1, dinv, W1, b1[None, :], W2)

    p = propf(xs2.reshape(2 * N, 64), src_f, dst_f)
    xs3 = _tc3(p, xs2, dinv, b2[None, :], W3)

    p = prop3(xs3.reshape(2 * N, 64), src_f, dst_f)
    xs4 = _tc4(p, xs3, dinv, b3[None, :], W4)

    p = prop4(xs4.reshape(4 * N, 32), src_q, dst_f)

    bg = (b_ih + b_hh)[None, :]
    out = _tc5(p, xs4, dinv, b4[None, :],
               batch.astype(jnp.int32)[:, None], W_ih.T, bg, W_fc,
               b_fc[None, :])
    return out
